# baseline (speedup 1.0000x reference)
import functools

import jax
import jax.numpy as jnp
from jax import lax
from jax.experimental import pallas as pl
from jax.experimental.pallas import tpu as pltpu
from jax.experimental.pallas import tpu_sc as plsc

N = 10000
E = 160000
D = 256
H1 = 256
H2 = 512

NC = 2
NS = 16
LANES = 16

N_PAD = 10240
ROWS_PER_TILE = N_PAD // NS
E_PAD = 163840
AGG_B = 128
AGG_ROWS = E_PAD // AGG_B
AGG_RPT = AGG_ROWS // NS
DEC_B = 64
DEC_ROWS = E_PAD // DEC_B
DEC_RPT = DEC_ROWS // (NC * NS)

ROW_BLK = 1280
TC_GRID = N_PAD // ROW_BLK

_MESH = plsc.VectorSubcoreMesh(core_axis_name="c", subcore_axis_name="s")
f32 = jnp.float32


DEG_RPT = AGG_ROWS // (NC * NS)


@functools.partial(
    pl.kernel,
    out_type=jax.ShapeDtypeStruct((NC, N_PAD, 128), f32),
    mesh=_MESH,
    scratch_types=[
        pltpu.VMEM((DEG_RPT, AGG_B), jnp.int32),
        pltpu.VMEM((AGG_B, 128), f32),
        pltpu.VMEM_SHARED((N_PAD, 128), f32),
    ],
)
def _deg_kernel(dst_hbm, ones_hbm, zeros_hbm, out_hbm, idx_v, ones_v, acc):
    c = lax.axis_index("c")
    s = lax.axis_index("s")
    wid = s * NC + c
    pltpu.sync_copy(dst_hbm.at[pl.ds(DEG_RPT * wid, DEG_RPT)], idx_v)
    pltpu.sync_copy(ones_hbm, ones_v)
    pltpu.sync_copy(zeros_hbm.at[pl.ds(ROWS_PER_TILE * s, ROWS_PER_TILE)],
                    acc.at[pl.ds(ROWS_PER_TILE * s, ROWS_PER_TILE)])
    plsc.subcore_barrier()

    def body(j, carry):
        pltpu.sync_copy(ones_v, acc.at[idx_v.at[j]], add=True)
        return carry

    lax.fori_loop(0, DEG_RPT, body, 0)
    plsc.subcore_barrier()
    pltpu.sync_copy(acc.at[pl.ds(ROWS_PER_TILE * s, ROWS_PER_TILE)],
                    out_hbm.at[c, pl.ds(ROWS_PER_TILE * s, ROWS_PER_TILE)])


@functools.partial(
    pl.kernel,
    out_type=(jax.ShapeDtypeStruct((N_PAD, 128), f32),
              jax.ShapeDtypeStruct((N_PAD, 128), f32)),
    mesh=_MESH,
    scratch_types=[
        pltpu.VMEM((AGG_RPT // 2, AGG_B), jnp.int32),
        pltpu.VMEM((AGG_RPT // 2, AGG_B), jnp.int32),
        pltpu.VMEM((AGG_B, 128), f32),
        pltpu.VMEM((AGG_B, 128), f32),
        pltpu.VMEM_SHARED((N_PAD, 128), f32),
        pltpu.SemaphoreType.DMA,
        pltpu.SemaphoreType.DMA,
    ],
)
def _agg_kernel(taba_hbm, tabb_hbm, src_hbm, dst_hbm, zeros_hbm,
                outa_hbm, outb_hbm, srcv, dstv, buf, buf2, acc, sem, sem2):
    c = lax.axis_index("c")
    s = lax.axis_index("s")
    HR = AGG_RPT // 2
    pltpu.sync_copy(zeros_hbm.at[pl.ds(ROWS_PER_TILE * s, ROWS_PER_TILE)],
                    acc.at[pl.ds(ROWS_PER_TILE * s, ROWS_PER_TILE)])
    plsc.subcore_barrier()

    def run(tab_hbm, out_hbm):
        bufs = (buf, buf2)
        sems = (sem, sem2)

        def gcopy(j, p):
            return pltpu.make_async_copy(tab_hbm.at[srcv.at[j]],
                                         bufs[p], sems[p])

        def half(h, carry):
            hb = pl.multiple_of(AGG_RPT * s + HR * h, 8)
            pltpu.sync_copy(src_hbm.at[pl.ds(hb, HR)], srcv)
            pltpu.sync_copy(dst_hbm.at[pl.ds(hb, HR)], dstv)
            gcopy(0, 0).start()

            def body(jj, carry2):
                j0 = 2 * jj
                gcopy(j0 + 1, 1).start()
                gcopy(j0, 0).wait()
                pltpu.sync_copy(buf, acc.at[dstv.at[j0]], add=True)

                @pl.when(jj < HR // 2 - 1)
                def _():
                    gcopy(j0 + 2, 0).start()

                gcopy(j0 + 1, 1).wait()
                pltpu.sync_copy(buf2, acc.at[dstv.at[j0 + 1]], add=True)
                return carry2

            lax.fori_loop(0, HR // 2, body, 0)
            return carry

        lax.fori_loop(0, 2, half, 0)
        plsc.subcore_barrier()
        pltpu.sync_copy(acc.at[pl.ds(ROWS_PER_TILE * s, ROWS_PER_TILE)],
                        out_hbm.at[pl.ds(ROWS_PER_TILE * s, ROWS_PER_TILE)])

    @pl.when(c == 0)
    def _():
        run(taba_hbm, outa_hbm)

    @pl.when(c == 1)
    def _():
        run(tabb_hbm, outb_hbm)


def _lane_gather(v, idx):
    return lax.gather(
        v, idx[:, None],
        lax.GatherDimensionNumbers(offset_dims=(), collapsed_slice_dims=(0,),
                                   start_index_map=(0,)),
        slice_sizes=(1,),
        mode=lax.GatherScatterMode.PROMISE_IN_BOUNDS)


DEC2_B = 32
DEC2_ROWS = E_PAD // DEC2_B
DEC2_RPT = DEC2_ROWS // (NC * NS)
bf16 = jnp.bfloat16


DEC2_CHUNK = E_PAD // (NC * NS)


@functools.partial(
    pl.kernel,
    out_type=(jax.ShapeDtypeStruct((E_PAD,), f32),
              jax.ShapeDtypeStruct((E_PAD,), f32)),
    mesh=_MESH,
    scratch_types=[
        pltpu.VMEM((DEC2_CHUNK,), jnp.int32),
        pltpu.VMEM((DEC2_CHUNK,), jnp.int32),
    ] + [pltpu.VMEM((DEC2_B, H2 // 2), jnp.int32) for _ in range(8)] + [
        pltpu.VMEM((DEC2_CHUNK,), f32),
        pltpu.VMEM((DEC2_CHUNK,), f32),
        pltpu.SemaphoreType.DMA,
        pltpu.SemaphoreType.DMA,
        pltpu.SemaphoreType.DMA,
        pltpu.SemaphoreType.DMA,
    ],
)
def _dec_kernel(zc_hbm, zn_hbm, src_hbm, dst_hbm, outc_hbm, outn_hbm,
                srcv, dstv, b0cs, b0cd, b0ns, b0nd, b1cs, b1cd, b1ns, b1nd,
                obc, obn, semc0, semc1, semn0, semn1):
    c = lax.axis_index("c")
    s = lax.axis_index("s")
    wid = s * NC + c
    base = DEC2_CHUNK * wid
    pltpu.sync_copy(src_hbm.at[pl.ds(base, DEC2_CHUNK)], srcv)
    pltpu.sync_copy(dst_hbm.at[pl.ds(base, DEC2_CHUNK)], dstv)
    lane = lax.iota(jnp.int32, LANES)
    perms = [jnp.bitwise_xor(lane, k) for k in (8, 4, 2, 1)]
    bufs = ((b0cs, b0cd, b0ns, b0nd), (b1cs, b1cd, b1ns, b1nd))
    sems = ((semc0, semn0), (semc1, semn1))

    def copies(j, p):
        bcs, bcd, bns, bnd = bufs[p]
        sc_, sn_ = sems[p]
        off = pl.multiple_of(DEC2_B * j, DEC2_B)
        si = srcv.at[pl.ds(off, DEC2_B)]
        di = dstv.at[pl.ds(off, DEC2_B)]
        return (pltpu.make_async_copy(zc_hbm.at[si], bcs, sc_),
                pltpu.make_async_copy(zc_hbm.at[di], bcd, sc_),
                pltpu.make_async_copy(zn_hbm.at[si], bns, sn_),
                pltpu.make_async_copy(zn_hbm.at[di], bnd, sn_))

    def fire(j, p):
        for cp in copies(j, p):
            cp.start()

    def compute_one(j, p, which):
        bcs, bcd, bns, bnd = bufs[p]
        cps = copies(j, p)
        if which == 0:
            cps[0].wait()
            cps[1].wait()
            groups = ((bcs, bcd, obc),)
        else:
            cps[2].wait()
            cps[3].wait()
            groups = ((bns, bnd, obn),)
        for bs_, bd_, ob in groups:
            for g in range(DEC2_B // LANES):
                def row(rr, tot, bs_=bs_, bd_=bd_, g=g):
                    r = g * LANES + rr
                    acc = jnp.zeros((LANES,), f32)
                    himask = jnp.full((LANES,), -65536, jnp.int32)
                    for ch in range(H2 // 32):
                        sw = bs_[r, pl.ds(ch * LANES, LANES)]
                        dw = bd_[r, pl.ds(ch * LANES, LANES)]
                        s0 = lax.bitcast_convert_type(
                            jnp.left_shift(sw, 16), f32)
                        s1 = lax.bitcast_convert_type(
                            jnp.bitwise_and(sw, himask), f32)
                        d0 = lax.bitcast_convert_type(
                            jnp.left_shift(dw, 16), f32)
                        d1 = lax.bitcast_convert_type(
                            jnp.bitwise_and(dw, himask), f32)
                        acc = acc + s0 * d0 + s1 * d1
                    for pm in perms:
                        acc = acc + _lane_gather(acc, pm)
                    return jnp.where(lane == rr, jnp.maximum(acc, 0.0), tot)

                tot = lax.fori_loop(0, LANES, row, jnp.zeros((LANES,), f32))
                ob[pl.ds(pl.multiple_of(DEC2_B * j + g * LANES, LANES),
                         LANES)] = tot

    NB = DEC2_CHUNK // DEC2_B
    fire(0, 0)

    def body(jj, carry):
        j0 = 2 * jj
        fire(j0 + 1, 1)
        compute_one(j0, 0, 0)
        compute_one(j0, 0, 1)

        @pl.when(jj < NB // 2 - 1)
        def _():
            fire(j0 + 2, 0)

        compute_one(j0 + 1, 1, 0)
        compute_one(j0 + 1, 1, 1)
        return carry

    lax.fori_loop(0, NB // 2, body, 0)
    pltpu.sync_copy(obc, outc_hbm.at[pl.ds(base, DEC2_CHUNK)])
    pltpu.sync_copy(obn, outn_hbm.at[pl.ds(base, DEC2_CHUNK)])


def _tc1_body(x_ref, w_ref, dinv_ref, oa_ref, ob_ref):
    t = jnp.dot(x_ref[...], w_ref[...], preferred_element_type=f32)
    t = t * dinv_ref[...]
    oa_ref[...] = t[:, :128]
    ob_ref[...] = t[:, 128:]


def _tc2_body(aa_ref, ab_ref, ta_ref, tb_ref, dinv_ref, b0_ref,
              oa_ref, ob_ref):
    dinv = dinv_ref[...]
    ha = dinv * jax.nn.relu(dinv * (aa_ref[...] + ta_ref[...])
                            + b0_ref[...][:, :128])
    hb = dinv * jax.nn.relu(dinv * (ab_ref[...] + tb_ref[...])
                            + b0_ref[...][:, 128:])
    oa_ref[...] = ha
    ob_ref[...] = hb


def _tc3_body(aa_ref, ab_ref, ha_ref, hb_ref, dinv_ref, w_ref, b_ref,
              epsc_ref, epsn_ref,
              muc_ref, mun_ref, lvc_ref, lvn_ref, zc_ref, zn_ref):
    dinv = dinv_ref[...]
    ga = dinv * (aa_ref[...] + ha_ref[...])
    gb = dinv * (ab_ref[...] + hb_ref[...])
    g = jnp.concatenate([ga, gb], axis=1)
    o = jnp.dot(g, w_ref[...], preferred_element_type=f32) + b_ref[...]
    muc = o[:, :H2]
    mun = o[:, H2:2 * H2]
    lvc = o[:, 2 * H2:3 * H2]
    lvn = o[:, 3 * H2:]
    muc_ref[...] = muc
    mun_ref[...] = mun
    lvc_ref[...] = lvc
    lvn_ref[...] = lvn
    zc_ref[...] = (muc + epsc_ref[...] * jnp.exp(0.5 * lvc)).astype(bf16)
    zn_ref[...] = (mun + epsn_ref[...] * jnp.exp(0.5 * lvn)).astype(bf16)


def _rows(i):
    return (i, 0)


def kernel(x, edge_index, W0, b0, Wmc, bmc, Wmn, bmn, Wlc, blc, Wln, bln,
           eps_c, eps_n):
    src, dst = edge_index[0], edge_index[1]
    pad = jnp.full((E_PAD - E,), N, jnp.int32)
    src_p = jnp.concatenate([src, pad])
    dst_p = jnp.concatenate([dst, pad])
    src2 = src_p.reshape(AGG_ROWS, AGG_B)
    dst2 = dst_p.reshape(AGG_ROWS, AGG_B)

    ones128 = jnp.zeros((AGG_B, 128), f32).at[:, 0].set(1.0)
    zeros128 = jnp.zeros((N_PAD, 128), f32)

    degp = _deg_kernel(dst2, ones128, zeros128)
    deg = degp[0, :, 0] + degp[1, :, 0] + 1.0
    dinv = jax.lax.rsqrt(deg)
    dinv = jnp.where(jnp.arange(N_PAD) < N, dinv, 0.0)
    dinv2 = dinv[:, None]

    x_p = jnp.pad(x, ((0, N_PAD - N), (0, 0)))

    t1a, t1b = pl.pallas_call(
        _tc1_body,
        grid=(TC_GRID,),
        in_specs=[
            pl.BlockSpec((ROW_BLK, D), _rows),
            pl.BlockSpec((D, H1), lambda i: (0, 0)),
            pl.BlockSpec((ROW_BLK, 1), _rows),
        ],
        out_specs=(pl.BlockSpec((ROW_BLK, 128), _rows),
                   pl.BlockSpec((ROW_BLK, 128), _rows)),
        out_shape=(jax.ShapeDtypeStruct((N_PAD, 128), f32),
                   jax.ShapeDtypeStruct((N_PAD, 128), f32)),
    )(x_p, W0, dinv2)

    a1a, a1b = _agg_kernel(t1a, t1b, src2, dst2, zeros128)

    b0r = b0[None, :]
    hpa, hpb = pl.pallas_call(
        _tc2_body,
        grid=(TC_GRID,),
        in_specs=[
            pl.BlockSpec((ROW_BLK, 128), _rows),
            pl.BlockSpec((ROW_BLK, 128), _rows),
            pl.BlockSpec((ROW_BLK, 128), _rows),
            pl.BlockSpec((ROW_BLK, 128), _rows),
            pl.BlockSpec((ROW_BLK, 1), _rows),
            pl.BlockSpec((1, H1), lambda i: (0, 0)),
        ],
        out_specs=(pl.BlockSpec((ROW_BLK, 128), _rows),
                   pl.BlockSpec((ROW_BLK, 128), _rows)),
        out_shape=(jax.ShapeDtypeStruct((N_PAD, 128), f32),
                   jax.ShapeDtypeStruct((N_PAD, 128), f32)),
    )(a1a, a1b, t1a, t1b, dinv2, b0r)

    a2a, a2b = _agg_kernel(hpa, hpb, src2, dst2, zeros128)

    Wcat = jnp.concatenate([Wmc, Wmn, Wlc, Wln], axis=1)
    bcat = jnp.concatenate([bmc, bmn, blc, bln])[None, :]
    epsc_p = jnp.pad(eps_c, ((0, N_PAD - N), (0, 0)))
    epsn_p = jnp.pad(eps_n, ((0, N_PAD - N), (0, 0)))
    outs = pl.pallas_call(
        _tc3_body,
        grid=(TC_GRID,),
        in_specs=[
            pl.BlockSpec((ROW_BLK, 128), _rows),
            pl.BlockSpec((ROW_BLK, 128), _rows),
            pl.BlockSpec((ROW_BLK, 128), _rows),
            pl.BlockSpec((ROW_BLK, 128), _rows),
            pl.BlockSpec((ROW_BLK, 1), _rows),
            pl.BlockSpec((H1, 4 * H2), lambda i: (0, 0)),
            pl.BlockSpec((1, 4 * H2), lambda i: (0, 0)),
            pl.BlockSpec((ROW_BLK, H2), _rows),
            pl.BlockSpec((ROW_BLK, H2), _rows),
        ],
        out_specs=tuple(pl.BlockSpec((ROW_BLK, H2), _rows) for _ in range(6)),
        out_shape=tuple(jax.ShapeDtypeStruct((N_PAD, H2), f32)
                        for _ in range(4))
        + tuple(jax.ShapeDtypeStruct((N_PAD, H2), bf16) for _ in range(2)),
    )(a2a, a2b, hpa, hpb, dinv2, Wcat, bcat, epsc_p, epsn_p)
    mu_c, mu_n, lv_c, lv_n, zc, zn = outs

    zc_i = lax.bitcast_convert_type(zc.reshape(N_PAD, H2 // 2, 2), jnp.int32)
    zn_i = lax.bitcast_convert_type(zn.reshape(N_PAD, H2 // 2, 2), jnp.int32)
    ewc2, ewn2 = _dec_kernel(zc_i, zn_i, src_p, dst_p)
    ew_c = ewc2[:E]
    ew_n = ewn2[:E]

    return (ew_c, ew_n, mu_c[:N], mu_n[:N], lv_c[:N], lv_n[:N])


# speedup vs baseline: 1.0082x; 1.0082x over previous
import functools

import jax
import jax.numpy as jnp
from jax import lax
from jax.experimental import pallas as pl
from jax.experimental.pallas import tpu as pltpu
from jax.experimental.pallas import tpu_sc as plsc

N = 10000
E = 160000
D = 256
H1 = 256
H2 = 512

NC = 2
NS = 16
LANES = 16

N_PAD = 10240
ROWS_PER_TILE = N_PAD // NS
E_PAD = 163840
AGG_B = 128
AGG_ROWS = E_PAD // AGG_B
AGG_RPT = AGG_ROWS // NS
DEC_B = 64
DEC_ROWS = E_PAD // DEC_B
DEC_RPT = DEC_ROWS // (NC * NS)

ROW_BLK = 1280
TC_GRID = N_PAD // ROW_BLK

_MESH = plsc.VectorSubcoreMesh(core_axis_name="c", subcore_axis_name="s")
f32 = jnp.float32


DEG_RPT = AGG_ROWS // (NC * NS)


@functools.partial(
    pl.kernel,
    out_type=jax.ShapeDtypeStruct((NC, N_PAD, 128), f32),
    mesh=_MESH,
    scratch_types=[
        pltpu.VMEM((DEG_RPT, AGG_B), jnp.int32),
        pltpu.VMEM((AGG_B, 128), f32),
        pltpu.VMEM_SHARED((N_PAD, 128), f32),
    ],
)
def _deg_kernel(dst_hbm, ones_hbm, zeros_hbm, out_hbm, idx_v, ones_v, acc):
    c = lax.axis_index("c")
    s = lax.axis_index("s")
    wid = s * NC + c
    pltpu.sync_copy(dst_hbm.at[pl.ds(DEG_RPT * wid, DEG_RPT)], idx_v)
    pltpu.sync_copy(ones_hbm, ones_v)
    pltpu.sync_copy(zeros_hbm.at[pl.ds(ROWS_PER_TILE * s, ROWS_PER_TILE)],
                    acc.at[pl.ds(ROWS_PER_TILE * s, ROWS_PER_TILE)])
    plsc.subcore_barrier()

    def body(j, carry):
        pltpu.sync_copy(ones_v, acc.at[idx_v.at[j]], add=True)
        return carry

    lax.fori_loop(0, DEG_RPT, body, 0)
    plsc.subcore_barrier()
    pltpu.sync_copy(acc.at[pl.ds(ROWS_PER_TILE * s, ROWS_PER_TILE)],
                    out_hbm.at[c, pl.ds(ROWS_PER_TILE * s, ROWS_PER_TILE)])


@functools.partial(
    pl.kernel,
    out_type=(jax.ShapeDtypeStruct((N_PAD, 128), f32),
              jax.ShapeDtypeStruct((N_PAD, 128), f32)),
    mesh=_MESH,
    scratch_types=[
        pltpu.VMEM((AGG_RPT // 2, AGG_B), jnp.int32),
        pltpu.VMEM((AGG_RPT // 2, AGG_B), jnp.int32),
        pltpu.VMEM((AGG_B, 128), f32),
        pltpu.VMEM((AGG_B, 128), f32),
        pltpu.VMEM_SHARED((N_PAD, 128), f32),
        pltpu.SemaphoreType.DMA,
        pltpu.SemaphoreType.DMA,
    ],
)
def _agg_kernel(taba_hbm, tabb_hbm, src_hbm, dst_hbm, zeros_hbm,
                outa_hbm, outb_hbm, srcv, dstv, buf, buf2, acc, sem, sem2):
    c = lax.axis_index("c")
    s = lax.axis_index("s")
    HR = AGG_RPT // 2
    pltpu.sync_copy(zeros_hbm.at[pl.ds(ROWS_PER_TILE * s, ROWS_PER_TILE)],
                    acc.at[pl.ds(ROWS_PER_TILE * s, ROWS_PER_TILE)])
    plsc.subcore_barrier()

    def run(tab_hbm, out_hbm):
        bufs = (buf, buf2)
        sems = (sem, sem2)

        def gcopy(j, p):
            return pltpu.make_async_copy(tab_hbm.at[srcv.at[j]],
                                         bufs[p], sems[p])

        def half(h, carry):
            hb = pl.multiple_of(AGG_RPT * s + HR * h, 8)
            pltpu.sync_copy(src_hbm.at[pl.ds(hb, HR)], srcv)
            pltpu.sync_copy(dst_hbm.at[pl.ds(hb, HR)], dstv)
            gcopy(0, 0).start()

            def body(jj, carry2):
                j0 = 2 * jj
                gcopy(j0 + 1, 1).start()
                gcopy(j0, 0).wait()
                pltpu.sync_copy(buf, acc.at[dstv.at[j0]], add=True)

                @pl.when(jj < HR // 2 - 1)
                def _():
                    gcopy(j0 + 2, 0).start()

                gcopy(j0 + 1, 1).wait()
                pltpu.sync_copy(buf2, acc.at[dstv.at[j0 + 1]], add=True)
                return carry2

            lax.fori_loop(0, HR // 2, body, 0)
            return carry

        lax.fori_loop(0, 2, half, 0)
        plsc.subcore_barrier()
        pltpu.sync_copy(acc.at[pl.ds(ROWS_PER_TILE * s, ROWS_PER_TILE)],
                        out_hbm.at[pl.ds(ROWS_PER_TILE * s, ROWS_PER_TILE)])

    @pl.when(c == 0)
    def _():
        run(taba_hbm, outa_hbm)

    @pl.when(c == 1)
    def _():
        run(tabb_hbm, outb_hbm)


def _lane_gather(v, idx):
    return lax.gather(
        v, idx[:, None],
        lax.GatherDimensionNumbers(offset_dims=(), collapsed_slice_dims=(0,),
                                   start_index_map=(0,)),
        slice_sizes=(1,),
        mode=lax.GatherScatterMode.PROMISE_IN_BOUNDS)


DEC2_B = 32
DEC2_ROWS = E_PAD // DEC2_B
DEC2_RPT = DEC2_ROWS // (NC * NS)
bf16 = jnp.bfloat16


DEC2_CHUNK = E_PAD // (NC * NS)


@functools.partial(
    pl.kernel,
    out_type=(jax.ShapeDtypeStruct((E_PAD,), f32),
              jax.ShapeDtypeStruct((E_PAD,), f32)),
    mesh=_MESH,
    scratch_types=[
        pltpu.VMEM((DEC2_CHUNK,), jnp.int32),
        pltpu.VMEM((DEC2_CHUNK,), jnp.int32),
    ] + [pltpu.VMEM((DEC2_B, H2 // 2), jnp.int32) for _ in range(8)] + [
        pltpu.VMEM((DEC2_CHUNK,), f32),
        pltpu.VMEM((DEC2_CHUNK,), f32),
        pltpu.SemaphoreType.DMA,
        pltpu.SemaphoreType.DMA,
    ],
)
def _dec_kernel(zc_hbm, zn_hbm, src_hbm, dst_hbm, outc_hbm, outn_hbm,
                srcv, dstv, b0cs, b0cd, b0ns, b0nd, b1cs, b1cd, b1ns, b1nd,
                obc, obn, sem0, sem1):
    c = lax.axis_index("c")
    s = lax.axis_index("s")
    wid = s * NC + c
    base = DEC2_CHUNK * wid
    pltpu.sync_copy(src_hbm.at[pl.ds(base, DEC2_CHUNK)], srcv)
    pltpu.sync_copy(dst_hbm.at[pl.ds(base, DEC2_CHUNK)], dstv)
    lane = lax.iota(jnp.int32, LANES)
    perms = [jnp.bitwise_xor(lane, k) for k in (8, 4, 2, 1)]
    bufs = ((b0cs, b0cd, b0ns, b0nd), (b1cs, b1cd, b1ns, b1nd))
    sems = (sem0, sem1)

    def copies(j, p):
        bcs, bcd, bns, bnd = bufs[p]
        off = pl.multiple_of(DEC2_B * j, DEC2_B)
        si = srcv.at[pl.ds(off, DEC2_B)]
        di = dstv.at[pl.ds(off, DEC2_B)]
        return (pltpu.make_async_copy(zc_hbm.at[si], bcs, sems[p]),
                pltpu.make_async_copy(zc_hbm.at[di], bcd, sems[p]),
                pltpu.make_async_copy(zn_hbm.at[si], bns, sems[p]),
                pltpu.make_async_copy(zn_hbm.at[di], bnd, sems[p]))

    def fire(j, p):
        for cp in copies(j, p):
            cp.start()

    def drain(j, p):
        for cp in copies(j, p):
            cp.wait()

    def compute(j, p):
        bcs, bcd, bns, bnd = bufs[p]
        for bs_, bd_, ob in ((bcs, bcd, obc), (bns, bnd, obn)):
            for g in range(DEC2_B // LANES):
                def row(rr, tot, bs_=bs_, bd_=bd_, g=g):
                    r = g * LANES + rr
                    acc = jnp.zeros((LANES,), f32)
                    himask = jnp.full((LANES,), -65536, jnp.int32)
                    for ch in range(H2 // 32):
                        sw = bs_[r, pl.ds(ch * LANES, LANES)]
                        dw = bd_[r, pl.ds(ch * LANES, LANES)]
                        s0 = lax.bitcast_convert_type(
                            jnp.left_shift(sw, 16), f32)
                        s1 = lax.bitcast_convert_type(
                            jnp.bitwise_and(sw, himask), f32)
                        d0 = lax.bitcast_convert_type(
                            jnp.left_shift(dw, 16), f32)
                        d1 = lax.bitcast_convert_type(
                            jnp.bitwise_and(dw, himask), f32)
                        acc = acc + s0 * d0 + s1 * d1
                    for pm in perms:
                        acc = acc + _lane_gather(acc, pm)
                    return jnp.where(lane == rr, jnp.maximum(acc, 0.0), tot)

                tot = lax.fori_loop(0, LANES, row, jnp.zeros((LANES,), f32))
                ob[pl.ds(pl.multiple_of(DEC2_B * j + g * LANES, LANES),
                         LANES)] = tot

    NB = DEC2_CHUNK // DEC2_B
    fire(0, 0)

    def body(jj, carry):
        j0 = 2 * jj
        fire(j0 + 1, 1)
        drain(j0, 0)
        compute(j0, 0)

        @pl.when(jj < NB // 2 - 1)
        def _():
            fire(j0 + 2, 0)

        drain(j0 + 1, 1)
        compute(j0 + 1, 1)
        return carry

    lax.fori_loop(0, NB // 2, body, 0)
    pltpu.sync_copy(obc, outc_hbm.at[pl.ds(base, DEC2_CHUNK)])
    pltpu.sync_copy(obn, outn_hbm.at[pl.ds(base, DEC2_CHUNK)])


def _tc1_body(x_ref, w_ref, dinv_ref, oa_ref, ob_ref):
    t = jnp.dot(x_ref[...], w_ref[...], preferred_element_type=f32)
    t = t * dinv_ref[...]
    oa_ref[...] = t[:, :128]
    ob_ref[...] = t[:, 128:]


def _tc2_body(aa_ref, ab_ref, ta_ref, tb_ref, dinv_ref, b0_ref,
              oa_ref, ob_ref):
    dinv = dinv_ref[...]
    ha = dinv * jax.nn.relu(dinv * (aa_ref[...] + ta_ref[...])
                            + b0_ref[...][:, :128])
    hb = dinv * jax.nn.relu(dinv * (ab_ref[...] + tb_ref[...])
                            + b0_ref[...][:, 128:])
    oa_ref[...] = ha
    ob_ref[...] = hb


def _tc3_body(aa_ref, ab_ref, ha_ref, hb_ref, dinv_ref, w_ref, b_ref,
              epsc_ref, epsn_ref,
              muc_ref, mun_ref, lvc_ref, lvn_ref, zc_ref, zn_ref):
    dinv = dinv_ref[...]
    ga = dinv * (aa_ref[...] + ha_ref[...])
    gb = dinv * (ab_ref[...] + hb_ref[...])
    g = jnp.concatenate([ga, gb], axis=1)
    o = jnp.dot(g, w_ref[...], preferred_element_type=f32) + b_ref[...]
    muc = o[:, :H2]
    mun = o[:, H2:2 * H2]
    lvc = o[:, 2 * H2:3 * H2]
    lvn = o[:, 3 * H2:]
    muc_ref[...] = muc
    mun_ref[...] = mun
    lvc_ref[...] = lvc
    lvn_ref[...] = lvn
    zc_ref[...] = (muc + epsc_ref[...] * jnp.exp(0.5 * lvc)).astype(bf16)
    zn_ref[...] = (mun + epsn_ref[...] * jnp.exp(0.5 * lvn)).astype(bf16)


def _rows(i):
    return (i, 0)


def kernel(x, edge_index, W0, b0, Wmc, bmc, Wmn, bmn, Wlc, blc, Wln, bln,
           eps_c, eps_n):
    src, dst = edge_index[0], edge_index[1]
    pad = jnp.full((E_PAD - E,), N, jnp.int32)
    src_p = jnp.concatenate([src, pad])
    dst_p = jnp.concatenate([dst, pad])
    src2 = src_p.reshape(AGG_ROWS, AGG_B)
    dst2 = dst_p.reshape(AGG_ROWS, AGG_B)

    ones128 = jnp.zeros((AGG_B, 128), f32).at[:, 0].set(1.0)
    zeros128 = jnp.zeros((N_PAD, 128), f32)

    degp = _deg_kernel(dst2, ones128, zeros128)
    deg = degp[0, :, 0] + degp[1, :, 0] + 1.0
    dinv = jax.lax.rsqrt(deg)
    dinv = jnp.where(jnp.arange(N_PAD) < N, dinv, 0.0)
    dinv2 = dinv[:, None]

    x_p = jnp.pad(x, ((0, N_PAD - N), (0, 0)))

    t1a, t1b = pl.pallas_call(
        _tc1_body,
        grid=(TC_GRID,),
        in_specs=[
            pl.BlockSpec((ROW_BLK, D), _rows),
            pl.BlockSpec((D, H1), lambda i: (0, 0)),
            pl.BlockSpec((ROW_BLK, 1), _rows),
        ],
        out_specs=(pl.BlockSpec((ROW_BLK, 128), _rows),
                   pl.BlockSpec((ROW_BLK, 128), _rows)),
        out_shape=(jax.ShapeDtypeStruct((N_PAD, 128), f32),
                   jax.ShapeDtypeStruct((N_PAD, 128), f32)),
    )(x_p, W0, dinv2)

    a1a, a1b = _agg_kernel(t1a, t1b, src2, dst2, zeros128)

    b0r = b0[None, :]
    hpa, hpb = pl.pallas_call(
        _tc2_body,
        grid=(TC_GRID,),
        in_specs=[
            pl.BlockSpec((ROW_BLK, 128), _rows),
            pl.BlockSpec((ROW_BLK, 128), _rows),
            pl.BlockSpec((ROW_BLK, 128), _rows),
            pl.BlockSpec((ROW_BLK, 128), _rows),
            pl.BlockSpec((ROW_BLK, 1), _rows),
            pl.BlockSpec((1, H1), lambda i: (0, 0)),
        ],
        out_specs=(pl.BlockSpec((ROW_BLK, 128), _rows),
                   pl.BlockSpec((ROW_BLK, 128), _rows)),
        out_shape=(jax.ShapeDtypeStruct((N_PAD, 128), f32),
                   jax.ShapeDtypeStruct((N_PAD, 128), f32)),
    )(a1a, a1b, t1a, t1b, dinv2, b0r)

    a2a, a2b = _agg_kernel(hpa, hpb, src2, dst2, zeros128)

    Wcat = jnp.concatenate([Wmc, Wmn, Wlc, Wln], axis=1)
    bcat = jnp.concatenate([bmc, bmn, blc, bln])[None, :]
    epsc_p = jnp.pad(eps_c, ((0, N_PAD - N), (0, 0)))
    epsn_p = jnp.pad(eps_n, ((0, N_PAD - N), (0, 0)))
    outs = pl.pallas_call(
        _tc3_body,
        grid=(TC_GRID,),
        in_specs=[
            pl.BlockSpec((ROW_BLK, 128), _rows),
            pl.BlockSpec((ROW_BLK, 128), _rows),
            pl.BlockSpec((ROW_BLK, 128), _rows),
            pl.BlockSpec((ROW_BLK, 128), _rows),
            pl.BlockSpec((ROW_BLK, 1), _rows),
            pl.BlockSpec((H1, 4 * H2), lambda i: (0, 0)),
            pl.BlockSpec((1, 4 * H2), lambda i: (0, 0)),
            pl.BlockSpec((ROW_BLK, H2), _rows),
            pl.BlockSpec((ROW_BLK, H2), _rows),
        ],
        out_specs=tuple(pl.BlockSpec((ROW_BLK, H2), _rows) for _ in range(6)),
        out_shape=tuple(jax.ShapeDtypeStruct((N_PAD, H2), f32)
                        for _ in range(4))
        + tuple(jax.ShapeDtypeStruct((N_PAD, H2), bf16) for _ in range(2)),
    )(a2a, a2b, hpa, hpb, dinv2, Wcat, bcat, epsc_p, epsn_p)
    mu_c, mu_n, lv_c, lv_n, zc, zn = outs

    zc_i = lax.bitcast_convert_type(zc.reshape(N_PAD, H2 // 2, 2), jnp.int32)
    zn_i = lax.bitcast_convert_type(zn.reshape(N_PAD, H2 // 2, 2), jnp.int32)
    ewc2, ewn2 = _dec_kernel(zc_i, zn_i, src_p, dst_p)
    ew_c = ewc2[:E]
    ew_n = ewn2[:E]

    return (ew_c, ew_n, mu_c[:N], mu_n[:N], lv_c[:N], lv_n[:N])
